# baseline (speedup 1.0000x reference)
import functools

import jax
import jax.numpy as jnp
from jax import lax
from jax.experimental import pallas as pl
from jax.experimental.pallas import tpu as pltpu
from jax.experimental.pallas import tpu_sc as plsc

_NUM_WORKERS = 32
_CH = 48
_LANES = 16


@jax.jit
def _unpool(x, idx):
    n, c, h, w = x.shape
    ow = 2 * w
    planes = n * c
    planes_per_worker = planes // _NUM_WORKERS
    chunks = h // _CH
    nt = planes_per_worker * chunks

    mesh = plsc.VectorSubcoreMesh(core_axis_name="c", subcore_axis_name="s")

    @functools.partial(
        pl.kernel,
        mesh=mesh,
        out_type=jax.ShapeDtypeStruct((n, c, 2 * h, ow), jnp.float32),
        scratch_types=[
            pltpu.VMEM((_CH, w), jnp.float32),
            pltpu.VMEM((_CH, w), jnp.float32),
            pltpu.VMEM((_CH, w), jnp.int32),
            pltpu.VMEM((_CH, w), jnp.int32),
            pltpu.VMEM((2 * _CH, ow), jnp.float32),
            pltpu.VMEM((2 * _CH, ow), jnp.float32),
            pltpu.SemaphoreType.DMA,
            pltpu.SemaphoreType.DMA,
            pltpu.SemaphoreType.DMA,
            pltpu.SemaphoreType.DMA,
        ],
        compiler_params=pltpu.CompilerParams(
            needs_layout_passes=False, use_tc_tiling_on_sc=False),
    )
    def body(x_hbm, idx_hbm, out_hbm, xv0, xv1, iv0, iv1, ov0, ov1,
             si0, si1, so0, so1):
        xv = (xv0, xv1)
        iv = (iv0, iv1)
        ov = (ov0, ov1)
        si = (si0, si1)
        so = (so0, so1)
        wid = lax.axis_index("s") * 2 + lax.axis_index("c")
        zero16 = jnp.zeros((_LANES,), jnp.float32)

        def locate(t):
            p = wid * planes_per_worker + t // chunks
            i0 = (t % chunks) * _CH
            return p // c, p % c, i0

        def start_in(t, b):
            nn, cc, i0 = locate(t)
            pltpu.async_copy(
                x_hbm.at[nn, cc, pl.ds(i0, _CH), :], xv[b], si[b])
            pltpu.async_copy(
                idx_hbm.at[nn, cc, pl.ds(i0, _CH), :], iv[b], si[b])

        def wait_in(b):
            pltpu.make_async_copy(
                x_hbm.at[0, 0, pl.ds(0, _CH), :], xv[b], si[b]).wait()
            pltpu.make_async_copy(
                idx_hbm.at[0, 0, pl.ds(0, _CH), :], iv[b], si[b]).wait()

        def wait_out(b):
            pltpu.make_async_copy(
                ov[b], out_hbm.at[0, 0, pl.ds(0, 2 * _CH), :], so[b]).wait()

        start_in(0, 0)
        start_in(1, 1)

        def group(g, carry):
            for b in range(2):
                t = g * 2 + b
                wait_in(b)

                @pl.when(g > 0)
                def _():
                    wait_out(b)

                nn, cc, i0 = locate(t)

                @plsc.parallel_loop(0, 2 * _CH, unroll=2)
                def zero_loop(r2):
                    for u in range(ow // _LANES):
                        ov[b][r2, pl.ds(u * _LANES, _LANES)] = zero16

                @plsc.parallel_loop(0, _CH, unroll=2)
                def row_loop(r):
                    r2 = 2 * r
                    b2 = (i0 + r) * 2 * ow
                    for u in range(w // _LANES):
                        ids = iv[b][r, pl.ds(u * _LANES, _LANES)]
                        rel = ids - b2
                        odd = rel >= ow
                        rows = jnp.where(odd, r2 + 1, r2)
                        cols = jnp.where(odd, rel - ow, rel)
                        vals = xv[b][r, pl.ds(u * _LANES, _LANES)]
                        plsc.store_scatter(ov[b], [rows, cols], vals)

                pltpu.async_copy(
                    ov[b], out_hbm.at[nn, cc, pl.ds(2 * i0, 2 * _CH), :],
                    so[b])

                @pl.when(t + 2 < nt)
                def _():
                    start_in(t + 2, b)
            return carry

        lax.fori_loop(0, nt // 2, group, 0)
        wait_out(0)
        wait_out(1)

    return body(x, idx)


def kernel(x, indices):
    return _unpool(x, indices)


# speedup vs baseline: 2.4615x; 2.4615x over previous
import functools

import jax
import jax.numpy as jnp
from jax import lax
from jax.experimental import pallas as pl
from jax.experimental.pallas import tpu as pltpu
from jax.experimental.pallas import tpu_sc as plsc

_NUM_WORKERS = 32
_CH = 48
_LANES = 16


@jax.jit
def _unpool(x, idx):
    n, c, h, w = x.shape
    ow = 2 * w
    planes = n * c
    planes_per_worker = planes // _NUM_WORKERS
    chunks = h // _CH
    nt = planes_per_worker * chunks

    mesh = plsc.VectorSubcoreMesh(core_axis_name="c", subcore_axis_name="s")

    @functools.partial(
        pl.kernel,
        mesh=mesh,
        out_type=jax.ShapeDtypeStruct((n, c, 2 * h, ow), jnp.float32),
        scratch_types=[
            pltpu.VMEM((_CH, w), jnp.float32),
            pltpu.VMEM((_CH, w), jnp.float32),
            pltpu.VMEM((_CH, w), jnp.int32),
            pltpu.VMEM((_CH, w), jnp.int32),
            pltpu.VMEM((2 * _CH, ow), jnp.float32),
            pltpu.VMEM((2 * _CH, ow), jnp.float32),
            pltpu.SemaphoreType.DMA,
            pltpu.SemaphoreType.DMA,
            pltpu.SemaphoreType.DMA,
            pltpu.SemaphoreType.DMA,
        ],
        compiler_params=pltpu.CompilerParams(needs_layout_passes=False),
    )
    def body(x_hbm, idx_hbm, out_hbm, xv0, xv1, iv0, iv1, ov0, ov1,
             si0, si1, so0, so1):
        xv = (xv0, xv1)
        iv = (iv0, iv1)
        ov = (ov0, ov1)
        si = (si0, si1)
        so = (so0, so1)
        wid = lax.axis_index("s") * 2 + lax.axis_index("c")
        zero16 = jnp.zeros((_LANES,), jnp.float32)

        def locate(t):
            p = wid * planes_per_worker + t // chunks
            i0 = (t % chunks) * _CH
            return p // c, p % c, i0

        def start_in(t, b):
            nn, cc, i0 = locate(t)
            pltpu.async_copy(
                x_hbm.at[nn, cc, pl.ds(i0, _CH), :], xv[b], si[b])
            pltpu.async_copy(
                idx_hbm.at[nn, cc, pl.ds(i0, _CH), :], iv[b], si[b])

        def wait_in(b):
            pltpu.make_async_copy(
                x_hbm.at[0, 0, pl.ds(0, _CH), :], xv[b], si[b]).wait()
            pltpu.make_async_copy(
                idx_hbm.at[0, 0, pl.ds(0, _CH), :], iv[b], si[b]).wait()

        def wait_out(b):
            for s in range(2):
                pltpu.make_async_copy(
                    ov[b].at[pl.ds(s * _CH, _CH)],
                    out_hbm.at[0, 0, pl.ds(0, _CH), :], so[b]).wait()

        start_in(0, 0)
        start_in(1, 1)

        def group(g, carry):
            for b in range(2):
                t = g * 2 + b
                wait_in(b)

                @pl.when(g > 0)
                def _():
                    wait_out(b)

                nn, cc, i0 = locate(t)

                for s in range(2):
                    rlo = s * (_CH // 2)
                    rhi = rlo + _CH // 2

                    @plsc.parallel_loop(2 * rlo, 2 * rhi, unroll=2)
                    def zero_loop(r2):
                        for u in range(ow // _LANES):
                            ov[b][r2, pl.ds(u * _LANES, _LANES)] = zero16

                    @plsc.parallel_loop(rlo, rhi, unroll=2)
                    def row_loop(r):
                        r2 = 2 * r
                        b2 = (i0 + r) * 2 * ow
                        for u in range(w // _LANES):
                            ids = iv[b][r, pl.ds(u * _LANES, _LANES)]
                            rel = ids - b2
                            odd = rel >= ow
                            rows = jnp.where(odd, r2 + 1, r2)
                            cols = jnp.where(odd, rel - ow, rel)
                            vals = xv[b][r, pl.ds(u * _LANES, _LANES)]
                            plsc.store_scatter(ov[b], [rows, cols], vals)

                    pltpu.async_copy(
                        ov[b].at[pl.ds(2 * rlo, _CH)],
                        out_hbm.at[nn, cc, pl.ds(2 * i0 + 2 * rlo, _CH), :],
                        so[b])

                @pl.when(t + 2 < nt)
                def _():
                    start_in(t + 2, b)
            return carry

        lax.fori_loop(0, nt // 2, group, 0)
        wait_out(0)
        wait_out(1)

    return body(x, idx)


def kernel(x, indices):
    return _unpool(x, indices)


# speedup vs baseline: 2.5646x; 1.0419x over previous
import functools

import jax
import jax.numpy as jnp
from jax import lax
from jax.experimental import pallas as pl
from jax.experimental.pallas import tpu as pltpu
from jax.experimental.pallas import tpu_sc as plsc

_NUM_WORKERS = 32
_CH = 48
_LANES = 16


@jax.jit
def _unpool(x, idx):
    n, c, h, w = x.shape
    ow = 2 * w
    planes = n * c
    planes_per_worker = planes // _NUM_WORKERS
    chunks = h // _CH
    nt = planes_per_worker * chunks

    mesh = plsc.VectorSubcoreMesh(core_axis_name="c", subcore_axis_name="s")

    @functools.partial(
        pl.kernel,
        mesh=mesh,
        out_type=jax.ShapeDtypeStruct((n, c, 2 * h, ow), jnp.float32),
        scratch_types=[
            pltpu.VMEM((_CH, w), jnp.float32),
            pltpu.VMEM((_CH, w), jnp.float32),
            pltpu.VMEM((_CH, w), jnp.int32),
            pltpu.VMEM((_CH, w), jnp.int32),
            pltpu.VMEM((2 * _CH, ow), jnp.float32),
            pltpu.VMEM((2 * _CH, ow), jnp.float32),
            pltpu.SemaphoreType.DMA,
            pltpu.SemaphoreType.DMA,
            pltpu.SemaphoreType.DMA,
            pltpu.SemaphoreType.DMA,
        ],
        compiler_params=pltpu.CompilerParams(needs_layout_passes=False),
    )
    def body(x_hbm, idx_hbm, out_hbm, xv0, xv1, iv0, iv1, ov0, ov1,
             si0, si1, so0, so1):
        xv = (xv0, xv1)
        iv = (iv0, iv1)
        ov = (ov0, ov1)
        si = (si0, si1)
        so = (so0, so1)
        wid = lax.axis_index("s") * 2 + lax.axis_index("c")
        zero16 = jnp.zeros((_LANES,), jnp.float32)

        def locate(t):
            p = wid * planes_per_worker + t // chunks
            i0 = (t % chunks) * _CH
            return p // c, p % c, i0

        def start_in(t, b):
            nn, cc, i0 = locate(t)
            pltpu.async_copy(
                x_hbm.at[nn, cc, pl.ds(i0, _CH), :], xv[b], si[b])
            pltpu.async_copy(
                idx_hbm.at[nn, cc, pl.ds(i0, _CH), :], iv[b], si[b])

        def wait_in(b):
            pltpu.make_async_copy(
                x_hbm.at[0, 0, pl.ds(0, _CH), :], xv[b], si[b]).wait()
            pltpu.make_async_copy(
                idx_hbm.at[0, 0, pl.ds(0, _CH), :], iv[b], si[b]).wait()

        def wait_out(b):
            pltpu.make_async_copy(
                ov[b], out_hbm.at[0, 0, pl.ds(0, 2 * _CH), :], so[b]).wait()

        start_in(0, 0)
        start_in(1, 1)

        def group(g, carry):
            for b in range(2):
                t = g * 2 + b
                wait_in(b)

                @pl.when(g > 0)
                def _():
                    wait_out(b)

                nn, cc, i0 = locate(t)

                @plsc.parallel_loop(0, 2 * _CH, unroll=4)
                def zero_loop(r2):
                    for u in range(ow // _LANES):
                        ov[b][r2, pl.ds(u * _LANES, _LANES)] = zero16

                @plsc.parallel_loop(0, _CH, unroll=4)
                def row_loop(r):
                    r2 = 2 * r
                    b2 = (i0 + r) * 2 * ow
                    for u in range(w // _LANES):
                        ids = iv[b][r, pl.ds(u * _LANES, _LANES)]
                        rel = ids - b2
                        odd = rel >= ow
                        rows = jnp.where(odd, r2 + 1, r2)
                        cols = jnp.where(odd, rel - ow, rel)
                        vals = xv[b][r, pl.ds(u * _LANES, _LANES)]
                        plsc.store_scatter(ov[b], [rows, cols], vals)

                pltpu.async_copy(
                    ov[b], out_hbm.at[nn, cc, pl.ds(2 * i0, 2 * _CH), :],
                    so[b])

                @pl.when(t + 2 < nt)
                def _():
                    start_in(t + 2, b)
            return carry

        lax.fori_loop(0, nt // 2, group, 0)
        wait_out(0)
        wait_out(1)

    return body(x, idx)


def kernel(x, indices):
    return _unpool(x, indices)
